# baseline (speedup 1.0000x reference)
import functools

import jax
import jax.numpy as jnp
from jax import lax
from jax.experimental import pallas as pl
from jax.experimental.pallas import tpu as pltpu
from jax.experimental.pallas import tpu_sc as plsc

DIM_ = 64
NEMB_ = 512
ROWS_ = 128 * 512
NHALF_ = 2
HROWS_ = ROWS_ // NHALF_
TC_BLOCK_ = 2048
NW_ = 32
ROWS_PER_W_ = HROWS_ // NW_
NBUF_ = 4
PAD_ = 128
SC_CHUNK_ = 128
N_CHUNKS_ = ROWS_PER_W_ // SC_CHUNK_


def _tc_body(x_ref, e_ref, idx_ref, dsum_ref):
    i = pl.program_id(0)
    x = x_ref[...]
    e = e_ref[...]
    xe = jnp.dot(x, e, preferred_element_type=jnp.float32)
    dist = (
        jnp.sum(x * x, axis=1, keepdims=True)
        - 2.0 * xe
        + jnp.sum(e * e, axis=0, keepdims=True)
    )
    m = jnp.min(dist, axis=1, keepdims=True)
    jl = lax.broadcasted_iota(jnp.int32, (1, NEMB_), 1).astype(jnp.float32)
    masked = jnp.where(dist == m, jl, float(NEMB_))
    idx_ref[...] = jnp.min(masked, axis=1).astype(jnp.int32)

    @pl.when(i == 0)
    def _():
        dsum_ref[0, 0] = 0.0

    dsum_ref[0, 0] += jnp.sum(m)


def _tc_call(flat, embed, half):
    grid = HROWS_ // TC_BLOCK_
    base = half * grid
    return pl.pallas_call(
        _tc_body,
        grid=(grid,),
        in_specs=[
            pl.BlockSpec((TC_BLOCK_, DIM_), lambda i: (base + i, 0)),
            pl.BlockSpec((DIM_, NEMB_), lambda i: (0, 0)),
        ],
        out_specs=[
            pl.BlockSpec((TC_BLOCK_,), lambda i: (i,)),
            pl.BlockSpec(memory_space=pltpu.SMEM, block_shape=(1, 1),
                         index_map=lambda i: (0, 0)),
        ],
        out_shape=[
            jax.ShapeDtypeStruct((HROWS_,), jnp.int32),
            jax.ShapeDtypeStruct((1, 1), jnp.float32),
        ],
        name=f"vq_dist_argmin_h{half}",
    )(flat, embed)


def _sc_gather_body(table_hbm, idx_hbm, out_hbm, table_sp, idx_v, *bufs):
    rows = bufs[0:NBUF_]
    gsem = bufs[NBUF_:2 * NBUF_]
    osem = bufs[2 * NBUF_:3 * NBUF_]
    nc = 2
    sid = lax.axis_index("s")
    wid = sid * nc + lax.axis_index("c")
    base = wid * ROWS_PER_W_

    @pl.when(sid == 0)
    def _():
        pltpu.sync_copy(table_hbm, table_sp)

    pltpu.sync_copy(idx_hbm.at[wid], idx_v)
    plsc.subcore_barrier()
    gcp = [None] * N_CHUNKS_
    ocp = [None] * N_CHUNKS_
    for c in range(min(NBUF_, N_CHUNKS_)):
        gcp[c] = pltpu.async_copy(table_sp.at[idx_v.at[c]], rows[c], gsem[c])
    for c in range(N_CHUNKS_):
        b = c % NBUF_
        gcp[c].wait()
        ocp[c] = pltpu.async_copy(
            rows[b], out_hbm.at[pl.ds(base + c * SC_CHUNK_, SC_CHUNK_)], osem[b])
        nxt = c + NBUF_
        if nxt < N_CHUNKS_:
            ocp[c].wait()
            gcp[nxt] = pltpu.async_copy(table_sp.at[idx_v.at[nxt]], rows[b], gsem[b])
    for c in range(max(0, N_CHUNKS_ - NBUF_), N_CHUNKS_):
        ocp[c].wait()


@functools.cache
def _sc_gather():
    return pl.kernel(
        _sc_gather_body,
        out_type=jax.ShapeDtypeStruct((HROWS_, PAD_), jnp.float32),
        mesh=plsc.VectorSubcoreMesh(core_axis_name="c", subcore_axis_name="s"),
        scratch_types=(
            [pltpu.VMEM_SHARED((NEMB_, PAD_), jnp.float32)]
            + [pltpu.VMEM((N_CHUNKS_, SC_CHUNK_), jnp.int32)]
            + [pltpu.VMEM((SC_CHUNK_, PAD_), jnp.float32) for _ in range(NBUF_)]
            + [pltpu.SemaphoreType.DMA for _ in range(2 * NBUF_)]
        ),
    )


def kernel(inp, embed):
    flat = inp.reshape(ROWS_, DIM_)
    table = jnp.concatenate(
        [embed.T, jnp.zeros((NEMB_, PAD_ - DIM_), jnp.float32)], axis=1)
    sc = _sc_gather()
    idxs, dsums, qs = [], [], []
    for h in range(NHALF_):
        idx_h, dsum_h = _tc_call(flat, embed, h)
        qp_h = sc(table, idx_h.reshape(NW_, N_CHUNKS_, SC_CHUNK_))
        idxs.append(idx_h)
        dsums.append(dsum_h[0, 0])
        qs.append(qp_h[:, :DIM_])
    quantize_st = jnp.concatenate(qs, axis=0).reshape(128, 1, NEMB_, DIM_)
    diff = (1.25 / (ROWS_ * DIM_)) * (dsums[0] + dsums[1])
    embed_ind = jnp.concatenate(idxs, axis=0).reshape(128, 1, NEMB_)
    return quantize_st, diff, embed_ind


# speedup vs baseline: 1.0076x; 1.0076x over previous
import functools

import jax
import jax.numpy as jnp
from jax import lax
from jax.experimental import pallas as pl
from jax.experimental.pallas import tpu as pltpu
from jax.experimental.pallas import tpu_sc as plsc

DIM_ = 64
NEMB_ = 512
ROWS_ = 128 * 512
NHALF_ = 2
HROWS_ = ROWS_ // NHALF_
TC_BLOCK_ = 2048
NW_ = 32
ROWS_PER_W_ = HROWS_ // NW_
NBUF_ = 4
PAD_ = 128
SC_CHUNK_ = 128
N_CHUNKS_ = ROWS_PER_W_ // SC_CHUNK_


def _tc_body(x_ref, e_ref, idx_ref, dsum_ref):
    i = pl.program_id(0)
    x = x_ref[...]
    e = e_ref[...]
    xe = jnp.dot(x, e, preferred_element_type=jnp.float32)
    dist = (
        jnp.sum(x * x, axis=1, keepdims=True)
        - 2.0 * xe
        + jnp.sum(e * e, axis=0, keepdims=True)
    )
    m = jnp.min(dist, axis=1, keepdims=True)
    jl = lax.broadcasted_iota(jnp.int32, (1, NEMB_), 1).astype(jnp.float32)
    masked = jnp.where(dist == m, jl, float(NEMB_))
    idx_ref[...] = jnp.min(masked, axis=1).astype(jnp.int32)

    @pl.when(i == 0)
    def _():
        dsum_ref[0, 0] = 0.0

    dsum_ref[0, 0] += jnp.sum(m)


def _tc_call(flat, embed, half):
    grid = HROWS_ // TC_BLOCK_
    base = half * grid
    return pl.pallas_call(
        _tc_body,
        grid=(grid,),
        in_specs=[
            pl.BlockSpec((TC_BLOCK_, DIM_), lambda i: (base + i, 0)),
            pl.BlockSpec((DIM_, NEMB_), lambda i: (0, 0)),
        ],
        out_specs=[
            pl.BlockSpec((TC_BLOCK_,), lambda i: (i,)),
            pl.BlockSpec(memory_space=pltpu.SMEM, block_shape=(1, 1),
                         index_map=lambda i: (0, 0)),
        ],
        out_shape=[
            jax.ShapeDtypeStruct((HROWS_,), jnp.int32),
            jax.ShapeDtypeStruct((1, 1), jnp.float32),
        ],
        name=f"vq_dist_argmin_h{half}",
    )(flat, embed)


def _sc_gather_body(table_hbm, idx_hbm, out_hbm, table_sp, idx_v, *bufs):
    rows = bufs[0:NBUF_]
    gsem = bufs[NBUF_:2 * NBUF_]
    osem = bufs[2 * NBUF_:3 * NBUF_]
    nc = 2
    sid = lax.axis_index("s")
    wid = sid * nc + lax.axis_index("c")
    base = wid * ROWS_PER_W_

    @pl.when(sid == 0)
    def _():
        pltpu.sync_copy(table_hbm, table_sp)

    pltpu.sync_copy(idx_hbm.at[wid], idx_v)
    plsc.subcore_barrier()
    gcp = [None] * N_CHUNKS_
    ocp = [None] * N_CHUNKS_
    for c in range(min(NBUF_, N_CHUNKS_)):
        gcp[c] = pltpu.async_copy(table_sp.at[idx_v.at[c]], rows[c], gsem[c])
    for c in range(N_CHUNKS_):
        b = c % NBUF_
        gcp[c].wait()
        ocp[c] = pltpu.async_copy(
            rows[b], out_hbm.at[pl.ds(base + c * SC_CHUNK_, SC_CHUNK_)], osem[b])
        nxt = c + NBUF_
        if nxt < N_CHUNKS_:
            ocp[c].wait()
            gcp[nxt] = pltpu.async_copy(table_sp.at[idx_v.at[nxt]], rows[b], gsem[b])
    for c in range(max(0, N_CHUNKS_ - NBUF_), N_CHUNKS_):
        ocp[c].wait()


@functools.cache
def _sc_gather():
    return pl.kernel(
        _sc_gather_body,
        out_type=jax.ShapeDtypeStruct((HROWS_, PAD_), jnp.float32),
        mesh=plsc.VectorSubcoreMesh(core_axis_name="c", subcore_axis_name="s"),
        scratch_types=(
            [pltpu.VMEM_SHARED((NEMB_, PAD_), jnp.float32)]
            + [pltpu.VMEM((N_CHUNKS_, SC_CHUNK_), jnp.int32)]
            + [pltpu.VMEM((SC_CHUNK_, PAD_), jnp.float32) for _ in range(NBUF_)]
            + [pltpu.SemaphoreType.DMA for _ in range(2 * NBUF_)]
        ),
    )


UNPAD_BLOCK_ = 4096


def _unpad_body(qp_ref, out_ref):
    out_ref[...] = qp_ref[...][:, :DIM_]


def _unpad_call(qp_h, half, acc=None):
    grid = HROWS_ // UNPAD_BLOCK_
    base = half * grid
    operands = [qp_h]
    in_specs = [pl.BlockSpec((UNPAD_BLOCK_, PAD_), lambda i: (i, 0))]
    aliases = {}
    if acc is not None:
        operands.append(acc)
        in_specs.append(pl.BlockSpec(memory_space=pltpu.MemorySpace.HBM))
        aliases = {1: 0}

        def body(qp_ref, _acc_ref, out_ref):
            _unpad_body(qp_ref, out_ref)
    else:
        body = _unpad_body
    return pl.pallas_call(
        body,
        grid=(grid,),
        in_specs=in_specs,
        out_specs=pl.BlockSpec((UNPAD_BLOCK_, DIM_), lambda i: (base + i, 0)),
        out_shape=jax.ShapeDtypeStruct((ROWS_, DIM_), jnp.float32),
        input_output_aliases=aliases,
        name=f"vq_unpad_h{half}",
    )(*operands)


def kernel(inp, embed):
    flat = inp.reshape(ROWS_, DIM_)
    table = jnp.concatenate(
        [embed.T, jnp.zeros((NEMB_, PAD_ - DIM_), jnp.float32)], axis=1)
    sc = _sc_gather()
    idx0, dsum0 = _tc_call(flat, embed, 0)
    qp0 = sc(table, idx0.reshape(NW_, N_CHUNKS_, SC_CHUNK_))
    idx1, dsum1 = _tc_call(flat, embed, 1)
    q = _unpad_call(qp0, 0)
    qp1 = sc(table, idx1.reshape(NW_, N_CHUNKS_, SC_CHUNK_))
    q = _unpad_call(qp1, 1, acc=q)
    quantize_st = q.reshape(128, 1, NEMB_, DIM_)
    diff = (1.25 / (ROWS_ * DIM_)) * (dsum0[0, 0] + dsum1[0, 0])
    embed_ind = jnp.concatenate([idx0, idx1], axis=0).reshape(128, 1, NEMB_)
    return quantize_st, diff, embed_ind


# speedup vs baseline: 1.0094x; 1.0018x over previous
import functools

import jax
import jax.numpy as jnp
from jax import lax
from jax.experimental import pallas as pl
from jax.experimental.pallas import tpu as pltpu
from jax.experimental.pallas import tpu_sc as plsc

DIM_ = 64
NEMB_ = 512
ROWS_ = 128 * 512
NHALF_ = 2
HROWS_ = ROWS_ // NHALF_
TC_BLOCK_ = 2048
NW_ = 32
ROWS_PER_W_ = HROWS_ // NW_
NBUF_ = 4
PAD_ = 128
SC_CHUNK_ = 128
N_CHUNKS_ = ROWS_PER_W_ // SC_CHUNK_


def _tc_body(x_ref, e_ref, idx_ref, dsum_ref):
    i = pl.program_id(0)
    x = x_ref[...]
    e = e_ref[...]
    xe = jnp.dot(x, e, preferred_element_type=jnp.float32)
    dist = (
        jnp.sum(x * x, axis=1, keepdims=True)
        - 2.0 * xe
        + jnp.sum(e * e, axis=0, keepdims=True)
    )
    m = jnp.min(dist, axis=1, keepdims=True)
    jl = lax.broadcasted_iota(jnp.int32, (1, NEMB_), 1).astype(jnp.float32)
    masked = jnp.where(dist == m, jl, float(NEMB_))
    idx_ref[...] = jnp.min(masked, axis=1).astype(jnp.int32)

    @pl.when(i == 0)
    def _():
        dsum_ref[0, 0] = 0.0

    dsum_ref[0, 0] += jnp.sum(m)


def _tc_call(flat, embed, half):
    grid = HROWS_ // TC_BLOCK_
    base = half * grid
    return pl.pallas_call(
        _tc_body,
        grid=(grid,),
        in_specs=[
            pl.BlockSpec((TC_BLOCK_, DIM_), lambda i: (base + i, 0)),
            pl.BlockSpec((DIM_, NEMB_), lambda i: (0, 0)),
        ],
        out_specs=[
            pl.BlockSpec((TC_BLOCK_,), lambda i: (i,)),
            pl.BlockSpec(memory_space=pltpu.SMEM, block_shape=(1, 1),
                         index_map=lambda i: (0, 0)),
        ],
        out_shape=[
            jax.ShapeDtypeStruct((HROWS_,), jnp.int32),
            jax.ShapeDtypeStruct((1, 1), jnp.float32),
        ],
        name=f"vq_dist_argmin_h{half}",
    )(flat, embed)


def _sc_gather_body(table_hbm, idx_hbm, out_hbm, table_sp, idx_v, *bufs):
    rows = bufs[0:NBUF_]
    gsem = bufs[NBUF_:2 * NBUF_]
    osem = bufs[2 * NBUF_:3 * NBUF_]
    nc = 2
    sid = lax.axis_index("s")
    wid = sid * nc + lax.axis_index("c")
    base = wid * ROWS_PER_W_

    @pl.when(sid == 0)
    def _():
        pltpu.sync_copy(table_hbm, table_sp)

    pltpu.sync_copy(idx_hbm.at[wid], idx_v)
    plsc.subcore_barrier()
    gcp = [None] * N_CHUNKS_
    ocp = [None] * N_CHUNKS_
    for c in range(min(NBUF_, N_CHUNKS_)):
        gcp[c] = pltpu.async_copy(table_sp.at[idx_v.at[c]], rows[c], gsem[c])
    for c in range(N_CHUNKS_):
        b = c % NBUF_
        gcp[c].wait()
        ocp[c] = pltpu.async_copy(
            rows[b], out_hbm.at[pl.ds(base + c * SC_CHUNK_, SC_CHUNK_)], osem[b])
        nxt = c + NBUF_
        if nxt < N_CHUNKS_:
            ocp[c].wait()
            gcp[nxt] = pltpu.async_copy(table_sp.at[idx_v.at[nxt]], rows[b], gsem[b])
    for c in range(max(0, N_CHUNKS_ - NBUF_), N_CHUNKS_):
        ocp[c].wait()


@functools.cache
def _sc_gather():
    return pl.kernel(
        _sc_gather_body,
        out_type=jax.ShapeDtypeStruct((HROWS_, PAD_), jnp.float32),
        mesh=plsc.VectorSubcoreMesh(core_axis_name="c", subcore_axis_name="s"),
        scratch_types=(
            [pltpu.VMEM_SHARED((NEMB_, PAD_), jnp.float32)]
            + [pltpu.VMEM((N_CHUNKS_, SC_CHUNK_), jnp.int32)]
            + [pltpu.VMEM((SC_CHUNK_, PAD_), jnp.float32) for _ in range(NBUF_)]
            + [pltpu.SemaphoreType.DMA for _ in range(2 * NBUF_)]
        ),
        compiler_params=pltpu.CompilerParams(use_tc_tiling_on_sc=True),
    )


UNPAD_BLOCK_ = 4096


def _unpad_body(qp_ref, out_ref):
    out_ref[...] = qp_ref[...][:, :DIM_]


def _unpad_call(qp_h, half, acc=None):
    grid = HROWS_ // UNPAD_BLOCK_
    base = half * grid
    operands = [qp_h]
    in_specs = [pl.BlockSpec((UNPAD_BLOCK_, PAD_), lambda i: (i, 0))]
    aliases = {}
    if acc is not None:
        operands.append(acc)
        in_specs.append(pl.BlockSpec(memory_space=pltpu.MemorySpace.HBM))
        aliases = {1: 0}

        def body(qp_ref, _acc_ref, out_ref):
            _unpad_body(qp_ref, out_ref)
    else:
        body = _unpad_body
    return pl.pallas_call(
        body,
        grid=(grid,),
        in_specs=in_specs,
        out_specs=pl.BlockSpec((UNPAD_BLOCK_, DIM_), lambda i: (base + i, 0)),
        out_shape=jax.ShapeDtypeStruct((ROWS_, DIM_), jnp.float32),
        input_output_aliases=aliases,
        name=f"vq_unpad_h{half}",
    )(*operands)


def kernel(inp, embed):
    flat = inp.reshape(ROWS_, DIM_)
    table = jnp.concatenate(
        [embed.T, jnp.zeros((NEMB_, PAD_ - DIM_), jnp.float32)], axis=1)
    sc = _sc_gather()
    idx0, dsum0 = _tc_call(flat, embed, 0)
    qp0 = sc(table, idx0.reshape(NW_, N_CHUNKS_, SC_CHUNK_))
    idx1, dsum1 = _tc_call(flat, embed, 1)
    q = _unpad_call(qp0, 0)
    qp1 = sc(table, idx1.reshape(NW_, N_CHUNKS_, SC_CHUNK_))
    q = _unpad_call(qp1, 1, acc=q)
    quantize_st = q.reshape(128, 1, NEMB_, DIM_)
    diff = (1.25 / (ROWS_ * DIM_)) * (dsum0[0, 0] + dsum1[0, 0])
    embed_ind = jnp.concatenate([idx0, idx1], axis=0).reshape(128, 1, NEMB_)
    return quantize_st, diff, embed_ind


# speedup vs baseline: 1.0743x; 1.0643x over previous
import functools

import jax
import jax.numpy as jnp
from jax import lax
from jax.experimental import pallas as pl
from jax.experimental.pallas import tpu as pltpu
from jax.experimental.pallas import tpu_sc as plsc

DIM_ = 64
NEMB_ = 512
NBATCH_ = 128
ROWS_ = NBATCH_ * NEMB_
TCB_ = 4
NW_ = 32
ROWS_PER_W_ = ROWS_ // NW_
NBUF_ = 4
PAD_ = 128
SC_CHUNK_ = 128
N_CHUNKS_ = ROWS_PER_W_ // SC_CHUNK_


def _tc_body(xt_ref, e_ref, idx_ref, dsum_ref):
    i = pl.program_id(0)
    e = e_ref[...]
    c = jnp.sum(e * e, axis=0, keepdims=True)
    jl = lax.broadcasted_iota(jnp.int32, (1, NEMB_), 1).astype(jnp.float32)
    for k in range(TCB_):
        p = xt_ref[pl.ds(k * DIM_, DIM_), :]
        xe = lax.dot_general(p, e, (((0,), (0,)), ((), ())),
                             preferred_element_type=jnp.float32)
        aa = jnp.transpose(jnp.sum(p * p, axis=0, keepdims=True))
        dist = aa - 2.0 * xe + c
        m = jnp.min(dist, axis=1, keepdims=True)
        masked = jnp.where(dist == m, jl, float(NEMB_))
        idx_ref[pl.ds(k * NEMB_, NEMB_)] = jnp.min(masked, axis=1).astype(jnp.int32)

        @pl.when((i == 0) & (k == 0))
        def _():
            dsum_ref[0, 0] = 0.0

        dsum_ref[0, 0] += jnp.sum(m)


def _tc_call(xt, embed):
    grid = NBATCH_ // TCB_
    return pl.pallas_call(
        _tc_body,
        grid=(grid,),
        in_specs=[
            pl.BlockSpec((TCB_ * DIM_, NEMB_), lambda i: (i, 0)),
            pl.BlockSpec((DIM_, NEMB_), lambda i: (0, 0)),
        ],
        out_specs=[
            pl.BlockSpec((TCB_ * NEMB_,), lambda i: (i,)),
            pl.BlockSpec(memory_space=pltpu.SMEM, block_shape=(1, 1),
                         index_map=lambda i: (0, 0)),
        ],
        out_shape=[
            jax.ShapeDtypeStruct((ROWS_,), jnp.int32),
            jax.ShapeDtypeStruct((1, 1), jnp.float32),
        ],
        name="vq_dist_argmin",
    )(xt, embed)


def _sc_gather_body(table_hbm, idx_hbm, out_hbm, table_sp, idx_v, *bufs):
    rows = bufs[0:NBUF_]
    gsem = bufs[NBUF_:2 * NBUF_]
    osem = bufs[2 * NBUF_:3 * NBUF_]
    nc = 2
    sid = lax.axis_index("s")
    wid = sid * nc + lax.axis_index("c")
    base = wid * ROWS_PER_W_

    @pl.when(sid == 0)
    def _():
        pltpu.sync_copy(table_hbm, table_sp)

    pltpu.sync_copy(idx_hbm.at[wid], idx_v)
    plsc.subcore_barrier()
    gcp = [None] * N_CHUNKS_
    ocp = [None] * N_CHUNKS_
    for c in range(min(NBUF_, N_CHUNKS_)):
        gcp[c] = pltpu.async_copy(table_sp.at[idx_v.at[c]], rows[c], gsem[c])
    for c in range(N_CHUNKS_):
        b = c % NBUF_
        gcp[c].wait()
        ocp[c] = pltpu.async_copy(
            rows[b], out_hbm.at[pl.ds(base + c * SC_CHUNK_, SC_CHUNK_)], osem[b])
        nxt = c + NBUF_
        if nxt < N_CHUNKS_:
            ocp[c].wait()
            gcp[nxt] = pltpu.async_copy(table_sp.at[idx_v.at[nxt]], rows[b], gsem[b])
    for c in range(max(0, N_CHUNKS_ - NBUF_), N_CHUNKS_):
        ocp[c].wait()


@functools.cache
def _sc_gather():
    return pl.kernel(
        _sc_gather_body,
        out_type=jax.ShapeDtypeStruct((ROWS_, PAD_), jnp.float32),
        mesh=plsc.VectorSubcoreMesh(core_axis_name="c", subcore_axis_name="s"),
        scratch_types=(
            [pltpu.VMEM_SHARED((NEMB_, PAD_), jnp.float32)]
            + [pltpu.VMEM((N_CHUNKS_, SC_CHUNK_), jnp.int32)]
            + [pltpu.VMEM((SC_CHUNK_, PAD_), jnp.float32) for _ in range(NBUF_)]
            + [pltpu.SemaphoreType.DMA for _ in range(2 * NBUF_)]
        ),
    )


def kernel(inp, embed):
    xt = jnp.transpose(inp, (0, 2, 1)).reshape(NBATCH_ * DIM_, NEMB_)
    idx, dsum = _tc_call(xt, embed)
    table = jnp.concatenate(
        [embed.T, jnp.zeros((NEMB_, PAD_ - DIM_), jnp.float32)], axis=1)
    qp = _sc_gather()(table, idx.reshape(NW_, N_CHUNKS_, SC_CHUNK_))
    quantize_st = qp[:, :DIM_].reshape(NBATCH_, 1, NEMB_, DIM_)
    diff = (1.25 / (ROWS_ * DIM_)) * dsum[0, 0]
    embed_ind = idx.reshape(NBATCH_, 1, NEMB_)
    return quantize_st, diff, embed_ind
